# baseline (speedup 1.0000x reference)
import functools

import jax
import jax.numpy as jnp
from jax import lax
from jax.experimental import pallas as pl
from jax.experimental.pallas import tpu as pltpu
from jax.experimental.pallas import tpu_sc as plsc

VOCAB = 1000
NTOK = 1024 * 50

_NC, _NS, _LANES = 2, 16, 16
_NW = _NC * _NS
_TB = 128
_NBLK = NTOK // _TB
_BPW_MAX = -(-_NBLK // _NW)
_NU = 8
_D0 = -(-VOCAB // 8)
_TPU = 16


def _lse_body(table_ref, out_ref):
    x = table_ref[...]
    m = jnp.max(x, axis=1, keepdims=True)
    s = jnp.sum(jnp.exp(x - m), axis=1, keepdims=True)
    out_ref[...] = m + jnp.log(s)


def _lse_tc(table):
    v = table.shape[0]
    return pl.pallas_call(
        _lse_body,
        out_shape=jax.ShapeDtypeStruct((v, 1), jnp.float32),
    )(table)


def _sc_gather_build():
    mesh = plsc.VectorSubcoreMesh(core_axis_name="c", subcore_axis_name="s")

    @functools.partial(
        pl.kernel,
        mesh=mesh,
        compiler_params=pltpu.CompilerParams(
            needs_layout_passes=False, use_tc_tiling_on_sc=True
        ),
        out_type=(
            jax.ShapeDtypeStruct((_D0, _NBLK, 8, _TB), jnp.float32),
            jax.ShapeDtypeStruct((_NW, _LANES), jnp.float32),
        ),
        scratch_types=[
            pltpu.VMEM((_TB,), jnp.int32),
            pltpu.VMEM((_TB,), jnp.int32),
            pltpu.VMEM((_TB,), jnp.int32),
            pltpu.VMEM((_TB,), jnp.float32),
            pltpu.VMEM((_TB, 128), jnp.float32),
            pltpu.VMEM((_TB, 128), jnp.float32),
            pltpu.VMEM((_TPU, 8, _TB), jnp.float32),
            pltpu.VMEM((_TPU, 8, _TB), jnp.float32),
            pltpu.VMEM((VOCAB,), jnp.float32),
            pltpu.VMEM((_LANES,), jnp.float32),
            pltpu.SemaphoreType.DMA,
            pltpu.SemaphoreType.DMA,
            pltpu.SemaphoreType.DMA,
            pltpu.SemaphoreType.DMA,
            pltpu.SemaphoreType.DMA,
        ],
    )
    def k(idx_hbm, tgt_hbm, lse_hbm, tflat_hbm,
          t0_hbm, t1_hbm, t2_hbm, t3_hbm, t4_hbm, t5_hbm, t6_hbm, t7_hbm,
          out_hbm, part_hbm,
          idx_v, tgt_v, lin_v, vals_v, s0, s1, d0, d1, lse_v, acc_v,
          gsem0, gsem1, wsem0, wsem1, vsem):
        tins = (t0_hbm, t1_hbm, t2_hbm, t3_hbm, t4_hbm, t5_hbm, t6_hbm,
                t7_hbm)
        srcs = (s0, s1)
        dsts = (d0, d1)
        gsems = (gsem0, gsem1)
        wsems = (wsem0, wsem1)
        ntiles = [_TPU] * (_NU - 1) + [_D0 - _TPU * (_NU - 1)]
        wid = lax.axis_index("s") * _NC + lax.axis_index("c")
        pltpu.sync_copy(lse_hbm, lse_v)
        acc_v[...] = jnp.zeros((_LANES,), jnp.float32)
        iotas = [lax.iota(jnp.int32, _LANES) + (lg * _LANES)
                 for lg in range(_TB // _LANES)]

        def start_gather(u, p, blk):
            pltpu.make_async_copy(
                tins[u].at[idx_v], srcs[p], gsems[p]
            ).start()

        def wait_gather(p):
            pltpu.make_async_copy(
                tins[0].at[idx_v], srcs[p], gsems[p]
            ).wait()

        def start_write(u, p, blk):
            nt = ntiles[u]
            pltpu.make_async_copy(
                dsts[p].at[pl.ds(0, nt)],
                out_hbm.at[pl.ds(u * _TPU, nt), blk],
                wsems[p],
            ).start()

        def wait_write(u, p, blk):
            nt = ntiles[u]
            pltpu.make_async_copy(
                dsts[p].at[pl.ds(0, nt)],
                out_hbm.at[pl.ds(u * _TPU, nt), 0],
                wsems[p],
            ).wait()

        def transpose(u, p):
            src = srcs[p]
            dst = dsts[p]

            def body_t(t, carry):
                for c8 in range(8):
                    col = t * 8 + c8
                    colv = jnp.full((_LANES,), col, jnp.int32)
                    for lg in range(_TB // _LANES):
                        vec = plsc.load_gather(src, [iotas[lg], colv])
                        dst.at[t][c8, pl.ds(lg * _LANES, _LANES)] = vec
                return carry

            lax.fori_loop(0, ntiles[u], body_t, 0)

        def do_block(blk):
            base = blk * _TB
            pltpu.sync_copy(idx_hbm.at[pl.ds(base, _TB)], idx_v)
            pltpu.sync_copy(tgt_hbm.at[pl.ds(base, _TB)], tgt_v)
            for lg in range(_TB // _LANES):
                sl = pl.ds(lg * _LANES, _LANES)
                lin_v[sl] = idx_v[sl] * VOCAB + tgt_v[sl]
            pltpu.make_async_copy(
                tflat_hbm.at[lin_v], vals_v, vsem
            ).start()

            start_gather(0, 0, blk)
            for u in range(_NU):
                p = u % 2
                wait_gather(p)
                if u + 1 < _NU:
                    start_gather(u + 1, 1 - p, blk)
                if u >= 2:
                    wait_write(u - 2, p, blk)
                transpose(u, p)
                start_write(u, p, blk)

            pltpu.make_async_copy(tflat_hbm.at[lin_v], vals_v, vsem).wait()
            for lg in range(_TB // _LANES):
                sl = pl.ds(lg * _LANES, _LANES)
                lse_g = plsc.load_gather(lse_v, [idx_v[sl]])
                acc_v[...] = acc_v[...] + (lse_g - vals_v[sl])
            wait_write(_NU - 2, 0, blk)
            wait_write(_NU - 1, 1, blk)

        def body_b(b, carry):
            blk = wid + b * _NW

            @pl.when(blk < _NBLK)
            def _():
                do_block(blk)

            return carry

        lax.fori_loop(0, _BPW_MAX, body_b, 0)
        pltpu.sync_copy(acc_v, part_hbm.at[wid])

    return k


_sc_gather = _sc_gather_build()


def _loss_body(part_ref, out_ref):
    out_ref[...] = jnp.sum(part_ref[...], keepdims=True).reshape(1, 1) * (
        1.0 / NTOK
    )


def _loss_tc(partials):
    return pl.pallas_call(
        _loss_body,
        out_shape=jax.ShapeDtypeStruct((1, 1), jnp.float32),
    )(partials)


def kernel(idx, targets, table):
    idx_f = idx.reshape(-1).astype(jnp.int32)
    tgt_f = targets.reshape(-1).astype(jnp.int32)
    lse = _lse_tc(table).reshape(VOCAB)
    tflat = table.reshape(-1)
    tslices = [table[:, u * 128:(u + 1) * 128] for u in range(_NU - 1)]
    tslices.append(
        jnp.pad(table[:, (_NU - 1) * 128:], ((0, 0), (0, _NU * 128 - VOCAB)))
    )
    out4d, partials = _sc_gather(idx_f, tgt_f, lse, tflat, *tslices)
    logits2d = out4d.transpose(1, 3, 0, 2).reshape(NTOK, VOCAB)
    loss = _loss_tc(partials)[0, 0]
    return (logits2d, loss)


# speedup vs baseline: 1.2058x; 1.2058x over previous
import functools

import jax
import jax.numpy as jnp
from jax import lax
from jax.experimental import pallas as pl
from jax.experimental.pallas import tpu as pltpu
from jax.experimental.pallas import tpu_sc as plsc

VOCAB = 1000
NTOK = 1024 * 50

_NC, _NS, _LANES = 2, 16, 16
_NW = _NC * _NS
_TB = 128
_NBLK = NTOK // _TB
_BPW_MAX = -(-_NBLK // _NW)
_NU = 8
_D0 = -(-VOCAB // 8)
_TPU = 16


def _lse_body(table_ref, out_ref):
    x = table_ref[...]
    m = jnp.max(x, axis=1, keepdims=True)
    s = jnp.sum(jnp.exp(x - m), axis=1, keepdims=True)
    out_ref[...] = m + jnp.log(s)


def _lse_tc(table):
    v = table.shape[0]
    return pl.pallas_call(
        _lse_body,
        out_shape=jax.ShapeDtypeStruct((v, 1), jnp.float32),
    )(table)


def _sc_gather_build():
    mesh = plsc.VectorSubcoreMesh(core_axis_name="c", subcore_axis_name="s")

    @functools.partial(
        pl.kernel,
        mesh=mesh,
        compiler_params=pltpu.CompilerParams(
            needs_layout_passes=False, use_tc_tiling_on_sc=True
        ),
        out_type=(
            jax.ShapeDtypeStruct((_D0 * _NBLK * 8 * _TB,), jnp.float32),
            jax.ShapeDtypeStruct((_NW, _LANES), jnp.float32),
        ),
        scratch_types=[
            pltpu.VMEM((_TB,), jnp.int32),
            pltpu.VMEM((_TB,), jnp.int32),
            pltpu.VMEM((_TB,), jnp.int32),
            pltpu.VMEM((_TB,), jnp.float32),
            pltpu.VMEM((_TB, 128), jnp.float32),
            pltpu.VMEM((_TB, 128), jnp.float32),
            pltpu.VMEM((_TPU * 8 * _TB,), jnp.float32),
            pltpu.VMEM((_TPU * 8 * _TB,), jnp.float32),
            pltpu.VMEM((VOCAB,), jnp.float32),
            pltpu.VMEM((_LANES,), jnp.float32),
            pltpu.SemaphoreType.DMA,
            pltpu.SemaphoreType.DMA,
            pltpu.SemaphoreType.DMA,
            pltpu.SemaphoreType.DMA,
            pltpu.SemaphoreType.DMA,
        ],
    )
    def k(idx_hbm, tgt_hbm, lse_hbm, tflat_hbm,
          t0_hbm, t1_hbm, t2_hbm, t3_hbm, t4_hbm, t5_hbm, t6_hbm, t7_hbm,
          out_hbm, part_hbm,
          idx_v, tgt_v, lin_v, vals_v, s0, s1, d0, d1, lse_v, acc_v,
          gsem0, gsem1, wsem0, wsem1, vsem):
        tins = (t0_hbm, t1_hbm, t2_hbm, t3_hbm, t4_hbm, t5_hbm, t6_hbm,
                t7_hbm)
        srcs = (s0, s1)
        dsts = (d0, d1)
        gsems = (gsem0, gsem1)
        wsems = (wsem0, wsem1)
        ntiles = [_TPU] * (_NU - 1) + [_D0 - _TPU * (_NU - 1)]
        ncg = [8] * (_NU - 1) + [7]
        wid = lax.axis_index("s") * _NC + lax.axis_index("c")
        pltpu.sync_copy(lse_hbm, lse_v)
        acc_v[...] = jnp.zeros((_LANES,), jnp.float32)
        lane = lax.iota(jnp.int32, _LANES)
        addr_cg = [(lane + cg * _LANES) * _TB for cg in range(8)]

        def start_gather(u, p):
            pltpu.make_async_copy(
                tins[u].at[idx_v], srcs[p], gsems[p]
            ).start()

        def wait_gather(p):
            pltpu.make_async_copy(
                tins[0].at[idx_v], srcs[p], gsems[p]
            ).wait()

        def start_write(u, p, blk):
            for t in range(ntiles[u]):
                pltpu.make_async_copy(
                    dsts[p].at[pl.ds(t * 1024, 1024)],
                    out_hbm.at[pl.ds(((u * _TPU + t) * _NBLK + blk) * 1024,
                                     1024)],
                    wsems[p],
                ).start()

        def wait_write(u, p):
            for t in range(ntiles[u]):
                pltpu.make_async_copy(
                    dsts[p].at[pl.ds(t * 1024, 1024)],
                    out_hbm.at[pl.ds(0, 1024)],
                    wsems[p],
                ).wait()

        def transpose(u, p):
            src = srcs[p]
            dst = dsts[p]
            n = ncg[u]

            def body_t(tok, carry):
                tokv = jnp.full((_LANES,), 0, jnp.int32) + tok
                for cg in range(n):
                    vec = src[tok, pl.ds(cg * _LANES, _LANES)]
                    plsc.store_scatter(dst, [addr_cg[cg] + tokv], vec)
                return carry

            lax.fori_loop(0, _TB, body_t, 0)

        def do_block(blk):
            base = blk * _TB
            pltpu.sync_copy(idx_hbm.at[pl.ds(base, _TB)], idx_v)
            pltpu.sync_copy(tgt_hbm.at[pl.ds(base, _TB)], tgt_v)
            for lg in range(_TB // _LANES):
                sl = pl.ds(lg * _LANES, _LANES)
                lin_v[sl] = idx_v[sl] * VOCAB + tgt_v[sl]
            pltpu.make_async_copy(
                tflat_hbm.at[lin_v], vals_v, vsem
            ).start()

            start_gather(0, 0)
            for u in range(_NU):
                p = u % 2
                wait_gather(p)
                if u + 1 < _NU:
                    start_gather(u + 1, 1 - p)
                if u >= 2:
                    wait_write(u - 2, p)
                transpose(u, p)
                start_write(u, p, blk)

            pltpu.make_async_copy(tflat_hbm.at[lin_v], vals_v, vsem).wait()
            for lg in range(_TB // _LANES):
                sl = pl.ds(lg * _LANES, _LANES)
                lse_g = plsc.load_gather(lse_v, [idx_v[sl]])
                acc_v[...] = acc_v[...] + (lse_g - vals_v[sl])
            wait_write(_NU - 2, 0)
            wait_write(_NU - 1, 1)

        def body_b(b, carry):
            blk = wid + b * _NW

            @pl.when(blk < _NBLK)
            def _():
                do_block(blk)

            return carry

        lax.fori_loop(0, _BPW_MAX, body_b, 0)
        pltpu.sync_copy(acc_v, part_hbm.at[wid])

    return k


_sc_gather = _sc_gather_build()


def _loss_body(part_ref, out_ref):
    out_ref[...] = jnp.sum(part_ref[...], keepdims=True).reshape(1, 1) * (
        1.0 / NTOK
    )


def _loss_tc(partials):
    return pl.pallas_call(
        _loss_body,
        out_shape=jax.ShapeDtypeStruct((1, 1), jnp.float32),
    )(partials)


def kernel(idx, targets, table):
    idx_f = idx.reshape(-1).astype(jnp.int32)
    tgt_f = targets.reshape(-1).astype(jnp.int32)
    lse = _lse_tc(table).reshape(VOCAB)
    tflat = table.reshape(-1)
    tslices = [table[:, u * 128:(u + 1) * 128] for u in range(_NU - 1)]
    tslices.append(
        jnp.pad(table[:, (_NU - 1) * 128:], ((0, 0), (0, _NU * 128 - VOCAB)))
    )
    out1d, partials = _sc_gather(idx_f, tgt_f, lse, tflat, *tslices)
    out4d = out1d.reshape(_D0, _NBLK, 8, _TB)
    logits2d = out4d.transpose(1, 3, 0, 2).reshape(NTOK, VOCAB)
    loss = _loss_tc(partials)[0, 0]
    return (logits2d, loss)


# speedup vs baseline: 2.9284x; 2.4287x over previous
import functools

import jax
import jax.numpy as jnp
from jax import lax
from jax.experimental import pallas as pl
from jax.experimental.pallas import tpu as pltpu
from jax.experimental.pallas import tpu_sc as plsc

VOCAB = 1000
NTOK = 1024 * 50

_NC, _NS, _LANES = 2, 16, 16
_NW = _NC * _NS
_TB = 128
_NBLK = NTOK // _TB
_BPW_MAX = -(-_NBLK // _NW)
_NU = 8
_D0 = -(-VOCAB // 8)
_TPU = 16


def _lse_body(table_ref, out_ref):
    x = table_ref[...]
    m = jnp.max(x, axis=1, keepdims=True)
    s = jnp.sum(jnp.exp(x - m), axis=1, keepdims=True)
    out_ref[...] = m + jnp.log(s)


def _lse_tc(table):
    v = table.shape[0]
    return pl.pallas_call(
        _lse_body,
        out_shape=jax.ShapeDtypeStruct((v, 1), jnp.float32),
    )(table)


def _sc_gather_build():
    mesh = plsc.VectorSubcoreMesh(core_axis_name="c", subcore_axis_name="s")

    @functools.partial(
        pl.kernel,
        mesh=mesh,
        compiler_params=pltpu.CompilerParams(
            needs_layout_passes=False, use_tc_tiling_on_sc=True
        ),
        out_type=(
            jax.ShapeDtypeStruct((_D0 * _NBLK * 8 * _TB,), jnp.float32),
            jax.ShapeDtypeStruct((_NW, _LANES), jnp.float32),
        ),
        scratch_types=[
            pltpu.VMEM((_TB,), jnp.int32),
            pltpu.VMEM((_TB,), jnp.int32),
            pltpu.VMEM((_TB,), jnp.int32),
            pltpu.VMEM((_TB,), jnp.float32),
            pltpu.VMEM((_TB, 128), jnp.float32),
            pltpu.VMEM((_TB, 128), jnp.float32),
            pltpu.VMEM((_TPU * 8 * _TB,), jnp.float32),
            pltpu.VMEM((_TPU * 8 * _TB,), jnp.float32),
            pltpu.VMEM((VOCAB,), jnp.float32),
            pltpu.VMEM((_LANES,), jnp.float32),
            pltpu.SemaphoreType.DMA,
            pltpu.SemaphoreType.DMA,
            pltpu.SemaphoreType.DMA,
            pltpu.SemaphoreType.DMA,
            pltpu.SemaphoreType.DMA,
        ],
    )
    def k(idx_hbm, tgt_hbm, lse_hbm, tflat_hbm,
          t0_hbm, t1_hbm, t2_hbm, t3_hbm, t4_hbm, t5_hbm, t6_hbm, t7_hbm,
          out_hbm, part_hbm,
          idx_v, tgt_v, lin_v, vals_v, s0, s1, d0, d1, lse_v, acc_v,
          gsem0, gsem1, wsem0, wsem1, vsem):
        tins = (t0_hbm, t1_hbm, t2_hbm, t3_hbm, t4_hbm, t5_hbm, t6_hbm,
                t7_hbm)
        srcs = (s0, s1)
        dsts = (d0, d1)
        gsems = (gsem0, gsem1)
        wsems = (wsem0, wsem1)
        ntiles = [_TPU] * (_NU - 1) + [_D0 - _TPU * (_NU - 1)]
        ncg = [8] * (_NU - 1) + [7]
        wid = lax.axis_index("s") * _NC + lax.axis_index("c")
        pltpu.sync_copy(lse_hbm, lse_v)
        acc_v[...] = jnp.zeros((_LANES,), jnp.float32)
        lane = lax.iota(jnp.int32, _LANES)
        zerov = lane * 0
        skew = [(lane + k) % _LANES for k in range(_LANES)]
        s_off = [sk * _TB + lane for sk in skew]
        d_off = [lane * _TB + sk for sk in skew]

        def start_gather(u, p):
            pltpu.make_async_copy(
                tins[u].at[idx_v], srcs[p], gsems[p]
            ).start()

        def wait_gather(p):
            pltpu.make_async_copy(
                tins[0].at[idx_v], srcs[p], gsems[p]
            ).wait()

        def start_write(u, p, blk):
            for t in range(ntiles[u]):
                pltpu.make_async_copy(
                    dsts[p].at[pl.ds(t * 1024, 1024)],
                    out_hbm.at[pl.ds(((u * _TPU + t) * _NBLK + blk) * 1024,
                                     1024)],
                    wsems[p],
                ).start()

        def wait_write(u, p):
            for t in range(ntiles[u]):
                pltpu.make_async_copy(
                    dsts[p].at[pl.ds(t * 1024, 1024)],
                    out_hbm.at[pl.ds(0, 1024)],
                    wsems[p],
                ).wait()

        def transpose(u, p):
            src = srcs[p]
            dst = dsts[p]
            n = ncg[u]

            def body_t(t0i, carry):
                for cg in range(n):
                    soff = zerov + (t0i * (_LANES * _TB) + cg * _LANES)
                    doff = zerov + (cg * (_LANES * _TB) + t0i * _LANES)
                    for kk in range(_LANES):
                        vec = plsc.load_gather(src, [zerov, s_off[kk] + soff])
                        plsc.store_scatter(dst, [d_off[kk] + doff], vec)
                return carry

            lax.fori_loop(0, _TB // _LANES, body_t, 0)

        def do_block(blk):
            base = blk * _TB
            pltpu.sync_copy(idx_hbm.at[pl.ds(base, _TB)], idx_v)
            pltpu.sync_copy(tgt_hbm.at[pl.ds(base, _TB)], tgt_v)
            for lg in range(_TB // _LANES):
                sl = pl.ds(lg * _LANES, _LANES)
                lin_v[sl] = idx_v[sl] * VOCAB + tgt_v[sl]
            pltpu.make_async_copy(
                tflat_hbm.at[lin_v], vals_v, vsem
            ).start()

            start_gather(0, 0)
            for u in range(_NU):
                p = u % 2
                wait_gather(p)
                if u + 1 < _NU:
                    start_gather(u + 1, 1 - p)
                if u >= 2:
                    wait_write(u - 2, p)
                transpose(u, p)
                start_write(u, p, blk)

            pltpu.make_async_copy(tflat_hbm.at[lin_v], vals_v, vsem).wait()
            for lg in range(_TB // _LANES):
                sl = pl.ds(lg * _LANES, _LANES)
                lse_g = plsc.load_gather(lse_v, [idx_v[sl]])
                acc_v[...] = acc_v[...] + (lse_g - vals_v[sl])
            wait_write(_NU - 2, 0)
            wait_write(_NU - 1, 1)

        def body_b(b, carry):
            blk = wid + b * _NW

            @pl.when(blk < _NBLK)
            def _():
                do_block(blk)

            return carry

        lax.fori_loop(0, _BPW_MAX, body_b, 0)
        pltpu.sync_copy(acc_v, part_hbm.at[wid])

    return k


_sc_gather = _sc_gather_build()


def _loss_body(part_ref, out_ref):
    out_ref[...] = jnp.sum(part_ref[...], keepdims=True).reshape(1, 1) * (
        1.0 / NTOK
    )


def _loss_tc(partials):
    return pl.pallas_call(
        _loss_body,
        out_shape=jax.ShapeDtypeStruct((1, 1), jnp.float32),
    )(partials)


def kernel(idx, targets, table):
    idx_f = idx.reshape(-1).astype(jnp.int32)
    tgt_f = targets.reshape(-1).astype(jnp.int32)
    lse = _lse_tc(table).reshape(VOCAB)
    tflat = table.reshape(-1)
    tslices = [table[:, u * 128:(u + 1) * 128] for u in range(_NU - 1)]
    tslices.append(
        jnp.pad(table[:, (_NU - 1) * 128:], ((0, 0), (0, _NU * 128 - VOCAB)))
    )
    out1d, partials = _sc_gather(idx_f, tgt_f, lse, tflat, *tslices)
    out4d = out1d.reshape(_D0, _NBLK, 8, _TB)
    logits2d = out4d.transpose(1, 3, 0, 2).reshape(NTOK, VOCAB)
    loss = _loss_tc(partials)[0, 0]
    return (logits2d, loss)


# speedup vs baseline: 3.9706x; 1.3559x over previous
import functools

import jax
import jax.numpy as jnp
from jax import lax
from jax.experimental import pallas as pl
from jax.experimental.pallas import tpu as pltpu
from jax.experimental.pallas import tpu_sc as plsc

VOCAB = 1000
VPAD = 1024
NTOK = 1024 * 50


def _lse_body(table_ref, out_ref):
    x = table_ref[...]
    m = jnp.max(x, axis=1, keepdims=True)
    s = jnp.sum(jnp.exp(x - m), axis=1, keepdims=True)
    out_ref[...] = m + jnp.log(s)


def _lse_tc(table):
    v = table.shape[0]
    return pl.pallas_call(
        _lse_body,
        out_shape=jax.ShapeDtypeStruct((v, 1), jnp.float32),
    )(table)


_NC, _NS, _LANES = 2, 16, 16
_NW = _NC * _NS
_BPW = NTOK // _NW
_NBUF = 4
_CHUNK = 16
_NCHUNK = _BPW // _CHUNK
_VCHUNK = 128
_NVCH = _BPW // _VCHUNK
_TAIL = _BPW - _NVCH * _VCHUNK


def _sc_gather_build():
    mesh = plsc.VectorSubcoreMesh(core_axis_name="c", subcore_axis_name="s")

    @functools.partial(
        pl.kernel,
        mesh=mesh,
        compiler_params=pltpu.CompilerParams(
            needs_layout_passes=False, use_tc_tiling_on_sc=True
        ),
        out_type=(
            jax.ShapeDtypeStruct((NTOK, VPAD), jnp.float32),
            jax.ShapeDtypeStruct((_NW, _LANES), jnp.float32),
        ),
        scratch_types=[
            pltpu.VMEM((_BPW,), jnp.int32),
            pltpu.VMEM((_BPW,), jnp.int32),
            pltpu.VMEM((_BPW,), jnp.int32),
            pltpu.VMEM((_BPW,), jnp.float32),
            pltpu.VMEM((_CHUNK, VPAD), jnp.float32),
            pltpu.VMEM((_CHUNK, VPAD), jnp.float32),
            pltpu.VMEM((_CHUNK, VPAD), jnp.float32),
            pltpu.VMEM((_CHUNK, VPAD), jnp.float32),
            pltpu.VMEM((VOCAB,), jnp.float32),
            pltpu.VMEM((_LANES,), jnp.float32),
            pltpu.SemaphoreType.DMA,
            pltpu.SemaphoreType.DMA,
            pltpu.SemaphoreType.DMA,
            pltpu.SemaphoreType.DMA,
            pltpu.SemaphoreType.DMA,
            pltpu.SemaphoreType.DMA,
            pltpu.SemaphoreType.DMA,
            pltpu.SemaphoreType.DMA,
            pltpu.SemaphoreType.DMA,
        ],
    )
    def k(idx_hbm, tgt_hbm, lse_hbm, tpad_hbm, tflat_hbm, out_hbm, part_hbm,
          idx_v, tgt_v, lin_v, vals_v, rows0, rows1, rows2, rows3,
          lse_v, acc_v,
          gsem0, gsem1, gsem2, gsem3, wsem0, wsem1, wsem2, wsem3, vsem):
        rows = (rows0, rows1, rows2, rows3)
        gsems = (gsem0, gsem1, gsem2, gsem3)
        wsems = (wsem0, wsem1, wsem2, wsem3)
        wid = lax.axis_index("s") * _NC + lax.axis_index("c")
        base = wid * _BPW
        pltpu.sync_copy(idx_hbm.at[pl.ds(base, _BPW)], idx_v)
        pltpu.sync_copy(tgt_hbm.at[pl.ds(base, _BPW)], tgt_v)
        pltpu.sync_copy(lse_hbm, lse_v)
        acc_v[...] = jnp.zeros((_LANES,), jnp.float32)

        def lin_body(j, carry):
            sl = pl.ds(j * _LANES, _LANES)
            lin_v[sl] = idx_v[sl] * VOCAB + tgt_v[sl]
            return carry

        lax.fori_loop(0, _BPW // _LANES, lin_body, 0)

        def vals_fire(c, carry):
            pltpu.make_async_copy(
                tflat_hbm.at[lin_v.at[pl.ds(c * _VCHUNK, _VCHUNK)]],
                vals_v.at[pl.ds(c * _VCHUNK, _VCHUNK)],
                vsem,
            ).start()
            return carry

        lax.fori_loop(0, _NVCH, vals_fire, 0)
        pltpu.make_async_copy(
            tflat_hbm.at[lin_v.at[pl.ds(_NVCH * _VCHUNK, _TAIL)]],
            vals_v.at[pl.ds(_NVCH * _VCHUNK, _TAIL)],
            vsem,
        ).start()

        def start_gather(g, p):
            pltpu.make_async_copy(
                tpad_hbm.at[idx_v.at[pl.ds(g * _CHUNK, _CHUNK)]],
                rows[p], gsems[p],
            ).start()

        def wait_gather(p):
            pltpu.make_async_copy(
                tpad_hbm.at[idx_v.at[pl.ds(0, _CHUNK)]], rows[p], gsems[p]
            ).wait()

        def start_write(g, p):
            pltpu.make_async_copy(
                rows[p], out_hbm.at[pl.ds(base + g * _CHUNK, _CHUNK)], wsems[p]
            ).start()

        def wait_write(p):
            pltpu.make_async_copy(
                rows[p], out_hbm.at[pl.ds(base, _CHUNK)], wsems[p]
            ).wait()

        start_gather(0, 0)
        start_gather(1, 1)
        start_gather(2, 2)
        wait_gather(0)
        start_write(0, 0)
        start_gather(3, 3)
        wait_gather(1)
        start_write(1, 1)

        def body(i, carry):
            for q in range(_NBUF):
                g = _NBUF * i + 2 + q
                pw = q
                pg = (2 + q) % _NBUF
                wait_write(pw)
                start_gather(g + 2, pw)
                wait_gather(pg)
                start_write(g, pg)
            return carry

        lax.fori_loop(0, (_NCHUNK - 4) // _NBUF, body, 0)

        wait_gather(2)
        start_write(_NCHUNK - 2, 2)
        wait_gather(3)
        start_write(_NCHUNK - 1, 3)

        def vals_drain(c, carry):
            pltpu.make_async_copy(
                tflat_hbm.at[lin_v.at[pl.ds(0, _VCHUNK)]],
                vals_v.at[pl.ds(c * _VCHUNK, _VCHUNK)],
                vsem,
            ).wait()
            return carry

        lax.fori_loop(0, _NVCH, vals_drain, 0)
        pltpu.make_async_copy(
            tflat_hbm.at[lin_v.at[pl.ds(0, _TAIL)]],
            vals_v.at[pl.ds(_NVCH * _VCHUNK, _TAIL)],
            vsem,
        ).wait()

        def loss_body(j, carry):
            sl = pl.ds(j * _LANES, _LANES)
            lse_g = plsc.load_gather(lse_v, [idx_v[sl]])
            acc_v[...] = acc_v[...] + (lse_g - vals_v[sl])
            return carry

        lax.fori_loop(0, _BPW // _LANES, loss_body, 0)

        for p in range(_NBUF):
            wait_write(p)
        pltpu.sync_copy(acc_v, part_hbm.at[wid])

    return k


_sc_gather = _sc_gather_build()


def _loss_body(part_ref, out_ref):
    out_ref[...] = jnp.sum(part_ref[...], keepdims=True).reshape(1, 1) * (
        1.0 / NTOK
    )


def _loss_tc(partials):
    return pl.pallas_call(
        _loss_body,
        out_shape=jax.ShapeDtypeStruct((1, 1), jnp.float32),
    )(partials)


def kernel(idx, targets, table):
    idx_f = idx.reshape(-1).astype(jnp.int32)
    tgt_f = targets.reshape(-1).astype(jnp.int32)
    lse = _lse_tc(table).reshape(VOCAB)
    tpad = jnp.pad(table, ((0, 0), (0, VPAD - VOCAB)))
    tflat = table.reshape(-1)
    out_pad, partials = _sc_gather(idx_f, tgt_f, lse, tpad, tflat)
    loss = _loss_tc(partials)[0, 0]
    return (out_pad[:, :VOCAB], loss)


# speedup vs baseline: 3.9821x; 1.0029x over previous
import functools

import jax
import jax.numpy as jnp
from jax import lax
from jax.experimental import pallas as pl
from jax.experimental.pallas import tpu as pltpu
from jax.experimental.pallas import tpu_sc as plsc

VOCAB = 1000
VPAD = 1024
NTOK = 1024 * 50


def _lse_body(table_ref, out_ref):
    x = table_ref[...]
    m = jnp.max(x, axis=1, keepdims=True)
    s = jnp.sum(jnp.exp(x - m), axis=1, keepdims=True)
    out_ref[...] = m + jnp.log(s)


def _lse_tc(table):
    v = table.shape[0]
    return pl.pallas_call(
        _lse_body,
        out_shape=jax.ShapeDtypeStruct((v, 1), jnp.float32),
    )(table)


_NC, _NS, _LANES = 2, 16, 16
_NW = _NC * _NS
_BPW = NTOK // _NW
_NBUF = 4
_CHUNK = 16
_NCHUNK = _BPW // _CHUNK
_VCHUNK = 128
_NVCH = _BPW // _VCHUNK
_TAIL = _BPW - _NVCH * _VCHUNK


def _sc_gather_build():
    mesh = plsc.VectorSubcoreMesh(core_axis_name="c", subcore_axis_name="s")

    @functools.partial(
        pl.kernel,
        mesh=mesh,
        compiler_params=pltpu.CompilerParams(
            needs_layout_passes=False, use_tc_tiling_on_sc=True
        ),
        out_type=(
            jax.ShapeDtypeStruct((NTOK, VPAD), jnp.float32),
            jax.ShapeDtypeStruct((_NW, _LANES), jnp.float32),
        ),
        scratch_types=[
            pltpu.VMEM((_BPW,), jnp.int32),
            pltpu.VMEM((_BPW,), jnp.int32),
            pltpu.VMEM((_CHUNK, VPAD), jnp.float32),
            pltpu.VMEM((_CHUNK, VPAD), jnp.float32),
            pltpu.VMEM((_CHUNK, VPAD), jnp.float32),
            pltpu.VMEM((_CHUNK, VPAD), jnp.float32),
            pltpu.VMEM((VOCAB,), jnp.float32),
            pltpu.VMEM((_LANES,), jnp.float32),
            pltpu.SemaphoreType.DMA,
            pltpu.SemaphoreType.DMA,
            pltpu.SemaphoreType.DMA,
            pltpu.SemaphoreType.DMA,
            pltpu.SemaphoreType.DMA,
            pltpu.SemaphoreType.DMA,
            pltpu.SemaphoreType.DMA,
            pltpu.SemaphoreType.DMA,
        ],
    )
    def k(idx_hbm, tgt_hbm, lse_hbm, tpad_hbm, out_hbm, part_hbm,
          idx_v, tgt_v, rows0, rows1, rows2, rows3,
          lse_v, acc_v,
          gsem0, gsem1, gsem2, gsem3, wsem0, wsem1, wsem2, wsem3):
        rows = (rows0, rows1, rows2, rows3)
        gsems = (gsem0, gsem1, gsem2, gsem3)
        wsems = (wsem0, wsem1, wsem2, wsem3)
        sid = lax.axis_index("s")
        wid = sid * _NC + lax.axis_index("c")
        base = wid * _BPW
        pltpu.sync_copy(idx_hbm.at[pl.ds(base, _BPW)], idx_v)
        pltpu.sync_copy(tgt_hbm.at[pl.ds(base, _BPW)], tgt_v)
        pltpu.sync_copy(lse_hbm, lse_v)
        acc_v[...] = jnp.zeros((_LANES,), jnp.float32)
        lane = lax.iota(jnp.int32, _LANES)

        def start_gather(g, p):
            pltpu.make_async_copy(
                tpad_hbm.at[idx_v.at[pl.ds(g * _CHUNK, _CHUNK)]],
                rows[p], gsems[p],
            ).start()

        def wait_gather(p):
            pltpu.make_async_copy(
                tpad_hbm.at[idx_v.at[pl.ds(0, _CHUNK)]], rows[p], gsems[p]
            ).wait()

        def start_write(g, p):
            pltpu.make_async_copy(
                rows[p], out_hbm.at[pl.ds(base + g * _CHUNK, _CHUNK)], wsems[p]
            ).start()

        def wait_write(p):
            pltpu.make_async_copy(
                rows[p], out_hbm.at[pl.ds(base, _CHUNK)], wsems[p]
            ).wait()

        def loss_step(g, p):
            sl = pl.ds(g * _CHUNK, _CHUNK)
            c = tgt_v[sl]
            val = plsc.load_gather(rows[p], [lane, c])
            lse_g = plsc.load_gather(lse_v, [idx_v[sl]])
            acc_v[...] = acc_v[...] + (lse_g - val)

        start_gather(0, 0)
        start_gather(1, 1)
        start_gather(2, 2)
        wait_gather(0)
        loss_step(0, 0)
        start_write(0, 0)
        start_gather(3, 3)
        wait_gather(1)
        loss_step(1, 1)
        start_write(1, 1)

        def body(i, carry):
            for q in range(_NBUF):
                g = _NBUF * i + 2 + q
                pw = q
                pg = (2 + q) % _NBUF
                wait_write(pw)
                start_gather(g + 2, pw)
                wait_gather(pg)
                loss_step(g, pg)
                start_write(g, pg)
            return carry

        lax.fori_loop(0, (_NCHUNK - 4) // _NBUF, body, 0)

        wait_gather(2)
        loss_step(_NCHUNK - 2, 2)
        start_write(_NCHUNK - 2, 2)
        wait_gather(3)
        loss_step(_NCHUNK - 1, 3)
        start_write(_NCHUNK - 1, 3)

        for p in range(_NBUF):
            wait_write(p)
        pltpu.sync_copy(acc_v, part_hbm.at[wid])

    return k


_sc_gather = _sc_gather_build()


def _loss_body(part_ref, out_ref):
    out_ref[...] = jnp.sum(part_ref[...], keepdims=True).reshape(1, 1) * (
        1.0 / NTOK
    )


def _loss_tc(partials):
    return pl.pallas_call(
        _loss_body,
        out_shape=jax.ShapeDtypeStruct((1, 1), jnp.float32),
    )(partials)


def kernel(idx, targets, table):
    idx_f = idx.reshape(-1).astype(jnp.int32)
    tgt_f = targets.reshape(-1).astype(jnp.int32)
    lse = _lse_tc(table).reshape(VOCAB)
    tpad = jnp.pad(table, ((0, 0), (0, VPAD - VOCAB)))
    out_pad, partials = _sc_gather(idx_f, tgt_f, lse, tpad)
    loss = _loss_tc(partials)[0, 0]
    return (out_pad[:, :VOCAB], loss)
